# baseline (speedup 1.0000x reference)
import jax
import jax.numpy as jnp
from jax.experimental import pallas as pl
from jax.experimental.pallas import tpu as pltpu


def _gcn_kernel(x_ref, adj_a_ref, adj_b_ref, w_ref, w2_ref, b_ref, o_ref,
                sup_ref):
    m = pl.program_id(1)

    @pl.when(m == 0)
    def _():
        sup_ref[...] = jnp.dot(
            x_ref[0], w_ref[...], preferred_element_type=jnp.float32
        )

    tm = o_ref.shape[1]
    nh = adj_a_ref.shape[2]
    x_tile = x_ref[0, pl.ds(m * tm, tm), :]
    agg = jnp.dot(adj_a_ref[0], sup_ref[:nh, :],
                  preferred_element_type=jnp.float32)
    agg += jnp.dot(adj_b_ref[0], sup_ref[nh:, :],
                   preferred_element_type=jnp.float32)
    res = jnp.dot(x_tile, w2_ref[...], preferred_element_type=jnp.float32)
    o_ref[0] = agg + res + b_ref[...]


def kernel(input, adj, weight, weight2, bias):
    B, N, IN = input.shape
    OUT = weight.shape[1]
    TM = 512
    NH = N // 2

    grid = (B, N // TM)
    out = pl.pallas_call(
        _gcn_kernel,
        grid=grid,
        in_specs=[
            pl.BlockSpec((1, N, IN), lambda b, m: (b, 0, 0)),
            pl.BlockSpec((1, TM, NH), lambda b, m: (b, m, 0)),
            pl.BlockSpec((1, TM, NH), lambda b, m: (b, m, 1)),
            pl.BlockSpec((IN, OUT), lambda b, m: (0, 0)),
            pl.BlockSpec((IN, OUT), lambda b, m: (0, 0)),
            pl.BlockSpec((1, OUT), lambda b, m: (0, 0)),
        ],
        out_specs=pl.BlockSpec((1, TM, OUT), lambda b, m: (b, m, 0)),
        out_shape=jax.ShapeDtypeStruct((B, N, OUT), jnp.float32),
        scratch_shapes=[pltpu.VMEM((N, OUT), jnp.float32)],
        compiler_params=pltpu.CompilerParams(
            dimension_semantics=("parallel", "arbitrary"),
        ),
    )(input, adj, adj, weight, weight2, bias.reshape(1, OUT))
    return out


# speedup vs baseline: 1.0140x; 1.0140x over previous
import jax
import jax.numpy as jnp
from jax.experimental import pallas as pl
from jax.experimental.pallas import tpu as pltpu


def _gcn_kernel(x_ref, adj_ref, w_ref, w2_ref, b_ref, o_ref, sup_ref):
    m = pl.program_id(1)

    @pl.when(m == 0)
    def _():
        sup_ref[...] = jnp.dot(
            x_ref[0], w_ref[...], preferred_element_type=jnp.float32
        )

    tm = o_ref.shape[1]
    x_tile = x_ref[0, pl.ds(m * tm, tm), :]
    agg = jnp.dot(adj_ref[0], sup_ref[...], preferred_element_type=jnp.float32)
    res = jnp.dot(x_tile, w2_ref[...], preferred_element_type=jnp.float32)
    o_ref[0] = agg + res + b_ref[...]


def kernel(input, adj, weight, weight2, bias):
    B, N, IN = input.shape
    OUT = weight.shape[1]
    TM = 512

    grid = (B, N // TM)
    out = pl.pallas_call(
        _gcn_kernel,
        grid=grid,
        in_specs=[
            pl.BlockSpec((1, N, IN), lambda b, m: (b, 0, 0)),
            pl.BlockSpec((1, TM, N), lambda b, m: (b, m, 0)),
            pl.BlockSpec((IN, OUT), lambda b, m: (0, 0)),
            pl.BlockSpec((IN, OUT), lambda b, m: (0, 0)),
            pl.BlockSpec((1, OUT), lambda b, m: (0, 0)),
        ],
        out_specs=pl.BlockSpec((1, TM, OUT), lambda b, m: (b, m, 0)),
        out_shape=jax.ShapeDtypeStruct((B, N, OUT), jnp.float32),
        scratch_shapes=[pltpu.VMEM((N, OUT), jnp.float32)],
        compiler_params=pltpu.CompilerParams(
            dimension_semantics=("parallel", "arbitrary"),
        ),
    )(input, adj, weight, weight2, bias.reshape(1, OUT))
    return out
